# baseline (speedup 1.0000x reference)
import functools

import jax
import jax.numpy as jnp
from jax.experimental import pallas as pl
from jax.experimental.pallas import tpu as pltpu

B, N, D, M, DEG = 256, 64, 256, 256, 16
BB = 8
BBN = BB * N


def _abd_kernel(ids_ref, abd_ref):
    ids = ids_ref[...]
    ids_t = jnp.tile(ids, (BB, 1))
    cmod = jax.lax.broadcasted_iota(jnp.int32, (BBN, BBN), 1) % N
    acc = jnp.zeros((BBN, BBN), jnp.float32)
    for j in range(DEG):
        acc += (ids_t[:, j : j + 1] == cmod).astype(jnp.float32)
    rblk = jax.lax.broadcasted_iota(jnp.int32, (BBN, BBN), 0) // N
    cblk = jax.lax.broadcasted_iota(jnp.int32, (BBN, BBN), 1) // N
    abd_ref[...] = jnp.where(rblk == cblk, acc * (1.0 / DEG), 0.0)


def _ln(x, g, b, eps=1e-5):
    mu = jnp.mean(x, axis=-1, keepdims=True)
    xc = x - mu
    var = jnp.mean(xc * xc, axis=-1, keepdims=True)
    return xc * jax.lax.rsqrt(var + eps) * g + b


def _main_kernel(x_ref, abd_ref, w1_ref, b1_ref, g1_ref, be1_ref,
                 w2_ref, b2_ref, w3_ref, b3_ref, g2_ref, be2_ref,
                 w4_ref, b4_ref, out_ref):
    bf = jnp.bfloat16
    x = x_ref[...].reshape(BBN, D).astype(bf)
    h = jnp.dot(x, w1_ref[...].astype(bf), preferred_element_type=jnp.float32)
    h = jnp.maximum(h + b1_ref[...], 0.0)
    h = _ln(h, g1_ref[...], be1_ref[...])
    t1 = jnp.dot(h.astype(bf), w2_ref[...].astype(bf),
                 preferred_element_type=jnp.float32) + b2_ref[...]
    t1b = t1.astype(bf)
    t2 = jnp.dot(abd_ref[...].astype(bf), t1b, preferred_element_type=jnp.float32)
    t3 = jnp.concatenate([t1b, t2.astype(bf)], axis=1)
    h2 = jnp.dot(t3, w3_ref[...].astype(bf), preferred_element_type=jnp.float32)
    h2 = jnp.maximum(h2 + b3_ref[...], 0.0)
    h2 = _ln(h2, g2_ref[...], be2_ref[...])
    h2 = h2.reshape(BB, N, M)
    logits = jnp.sum(h2 * w4_ref[...], axis=2) + b4_ref[0, 0]
    lg = logits - jnp.max(logits, axis=1, keepdims=True)
    e = jnp.exp(lg)
    out_ref[...] = e / jnp.sum(e, axis=1, keepdims=True)


def kernel(inp, ids_list, W1, b1, g1, be1, W2, b2, W3, b3, g2, be2, W4, b4):
    ids32 = ids_list.astype(jnp.int32)
    abd = pl.pallas_call(
        _abd_kernel,
        out_shape=jax.ShapeDtypeStruct((BBN, BBN), jnp.float32),
    )(ids32)

    row = lambda v: v.reshape(1, -1)
    w4r = W4.reshape(1, 1, M)
    b4r = b4.reshape(1, 1)

    full = lambda s: pl.BlockSpec(s, lambda i: (0,) * len(s))
    out = pl.pallas_call(
        _main_kernel,
        grid=(B // BB,),
        in_specs=[
            pl.BlockSpec((BB, N, D), lambda i: (i, 0, 0)),
            full((BBN, BBN)),
            full((D, M)), full((1, M)), full((1, M)), full((1, M)),
            full((M, M)), full((1, M)),
            full((2 * M, M)), full((1, M)), full((1, M)), full((1, M)),
            full((1, 1, M)), full((1, 1)),
        ],
        out_specs=pl.BlockSpec((BB, N), lambda i: (i, 0)),
        out_shape=jax.ShapeDtypeStruct((B, N), jnp.float32),
    )(inp, abd, W1, row(b1), row(g1), row(be1), W2, row(b2),
      W3, row(b3), row(g2), row(be2), w4r, b4r)
    return out


# speedup vs baseline: 1.1712x; 1.1712x over previous
import jax
import jax.numpy as jnp
from jax.experimental import pallas as pl
from jax.experimental.pallas import tpu as pltpu

B, N, D, M, DEG = 256, 64, 256, 256, 16
BB = 8
BBN = BB * N


def _abd_kernel(ids_ref, abd_ref):
    ids = ids_ref[...]
    ids_t = jnp.tile(ids, (BB, 1))
    cmod = jax.lax.broadcasted_iota(jnp.int32, (BBN, BBN), 1) % N
    acc = jnp.zeros((BBN, BBN), jnp.float32)
    for j in range(DEG):
        acc += (ids_t[:, j : j + 1] == cmod).astype(jnp.float32)
    rblk = jax.lax.broadcasted_iota(jnp.int32, (BBN, BBN), 0) // N
    cblk = jax.lax.broadcasted_iota(jnp.int32, (BBN, BBN), 1) // N
    abd_ref[...] = jnp.where(rblk == cblk, acc * (1.0 / DEG), 0.0)


def _ln(x, g, b, eps=1e-5):
    mu = jnp.mean(x, axis=-1, keepdims=True)
    ms = jnp.mean(x * x, axis=-1, keepdims=True)
    a = jax.lax.rsqrt(ms - mu * mu + eps)
    return (x - mu) * a * g + b


def _main_kernel(x_ref, abd_ref, w1_ref, b1_ref, g1_ref, be1_ref,
                 w2_ref, b2_ref, w3a_ref, w3b_ref, b3_ref, g2_ref, be2_ref,
                 w4_ref, out_ref):
    x = x_ref[...].reshape(BBN, D)
    h = jnp.dot(x, w1_ref[...], preferred_element_type=jnp.float32)
    h = jnp.maximum(h + b1_ref[...], 0.0)
    h = _ln(h, g1_ref[...], be1_ref[...])
    t1 = jnp.dot(h, w2_ref[...], preferred_element_type=jnp.float32) + b2_ref[...]
    u = jnp.dot(t1, w3a_ref[...], preferred_element_type=jnp.float32)
    v = jnp.dot(t1, w3b_ref[...], preferred_element_type=jnp.float32)
    w = jnp.dot(abd_ref[...], v, preferred_element_type=jnp.float32)
    h2 = jnp.maximum(u + w + b3_ref[...], 0.0)
    h2 = _ln(h2, g2_ref[...], be2_ref[...])
    out_ref[...] = jnp.sum(h2 * w4_ref[...], axis=1, keepdims=True)


def _softmax_kernel(lg_ref, out_ref):
    lg = lg_ref[...]
    e = jnp.exp(lg - jnp.max(lg, axis=1, keepdims=True))
    out_ref[...] = e / jnp.sum(e, axis=1, keepdims=True)


def kernel(inp, ids_list, W1, b1, g1, be1, W2, b2, W3, b3, g2, be2, W4, b4):
    ids32 = ids_list.astype(jnp.int32)
    abd = pl.pallas_call(
        _abd_kernel,
        out_shape=jax.ShapeDtypeStruct((BBN, BBN), jnp.float32),
    )(ids32)

    row = lambda v: v.reshape(1, -1)
    full = lambda s: pl.BlockSpec(s, lambda i: (0,) * len(s))
    logits = pl.pallas_call(
        _main_kernel,
        grid=(B // BB,),
        in_specs=[
            pl.BlockSpec((BB, N, D), lambda i: (i, 0, 0)),
            full((BBN, BBN)),
            full((D, M)), full((1, M)), full((1, M)), full((1, M)),
            full((M, M)), full((1, M)),
            full((M, M)), full((M, M)), full((1, M)), full((1, M)), full((1, M)),
            full((1, M)),
        ],
        out_specs=pl.BlockSpec((BBN, 1), lambda i: (i, 0)),
        out_shape=jax.ShapeDtypeStruct((B * N, 1), jnp.float32),
    )(inp, abd, W1, row(b1), row(g1), row(be1), W2, row(b2),
      W3[:M], W3[M:], row(b3), row(g2), row(be2), W4.reshape(1, M))

    out = pl.pallas_call(
        _softmax_kernel,
        out_shape=jax.ShapeDtypeStruct((B, N), jnp.float32),
    )(logits.reshape(B, N))
    return out


# speedup vs baseline: 1.1713x; 1.0001x over previous
import jax
import jax.numpy as jnp
from jax.experimental import pallas as pl
from jax.experimental.pallas import tpu as pltpu

B, N, D, M, DEG = 256, 64, 256, 256, 16
BB = 8
BBN = BB * N


def _abd_kernel(ids_ref, abd_ref):
    ids = ids_ref[...]
    ids_t = jnp.tile(ids, (BB, 1))
    cmod = jax.lax.broadcasted_iota(jnp.int32, (BBN, BBN), 1) % N
    acc = jnp.zeros((BBN, BBN), jnp.float32)
    for j in range(DEG):
        acc += (ids_t[:, j : j + 1] == cmod).astype(jnp.float32)
    rblk = jax.lax.broadcasted_iota(jnp.int32, (BBN, BBN), 0) // N
    cblk = jax.lax.broadcasted_iota(jnp.int32, (BBN, BBN), 1) // N
    abd_ref[...] = jnp.where(rblk == cblk, acc * (1.0 / DEG), 0.0)


def _ln(x, g, b, eps=1e-5):
    mu = jnp.mean(x, axis=-1, keepdims=True)
    ms = jnp.mean(x * x, axis=-1, keepdims=True)
    a = jax.lax.rsqrt(ms - mu * mu + eps)
    return (x - mu) * a * g + b


def _main_kernel(x_ref, abd_ref, w1_ref, b1_ref, g1_ref, be1_ref,
                 w2_ref, b2_ref, w3a_ref, w3b_ref, b3_ref, g2_ref, be2_ref,
                 w4_ref, out_ref):
    x = x_ref[...].reshape(BBN, D)
    h = jnp.dot(x, w1_ref[...], preferred_element_type=jnp.float32)
    h = jnp.maximum(h + b1_ref[...], 0.0)
    h = _ln(h, g1_ref[...], be1_ref[...])
    t1 = jnp.dot(h, w2_ref[...], preferred_element_type=jnp.float32) + b2_ref[...]
    u = jnp.dot(t1, w3a_ref[...], preferred_element_type=jnp.float32)
    v = jnp.dot(t1, w3b_ref[...], preferred_element_type=jnp.float32)
    w = jnp.dot(abd_ref[...], v, preferred_element_type=jnp.float32)
    h2 = jnp.maximum(u + w + b3_ref[...], 0.0)
    h2 = _ln(h2, g2_ref[...], be2_ref[...])
    out_ref[...] = jnp.sum(h2 * w4_ref[...], axis=1, keepdims=True)


def _softmax_kernel(lg_ref, out_ref):
    lg = lg_ref[...]
    e = jnp.exp(lg - jnp.max(lg, axis=1, keepdims=True))
    out_ref[...] = e / jnp.sum(e, axis=1, keepdims=True)


def kernel(inp, ids_list, W1, b1, g1, be1, W2, b2, W3, b3, g2, be2, W4, b4):
    ids32 = ids_list.astype(jnp.int32)
    abd = pl.pallas_call(
        _abd_kernel,
        out_shape=jax.ShapeDtypeStruct((BBN, BBN), jnp.float32),
    )(ids32)

    row = lambda v: v.reshape(1, -1)
    full = lambda s: pl.BlockSpec(s, lambda i: (0,) * len(s))
    logits = pl.pallas_call(
        _main_kernel,
        grid=(B // BB,),
        in_specs=[
            pl.BlockSpec((BB, N, D), lambda i: (i, 0, 0)),
            full((BBN, BBN)),
            full((D, M)), full((1, M)), full((1, M)), full((1, M)),
            full((M, M)), full((1, M)),
            full((M, M)), full((M, M)), full((1, M)), full((1, M)), full((1, M)),
            full((1, M)),
        ],
        out_specs=pl.BlockSpec((BBN, 1), lambda i: (i, 0)),
        out_shape=jax.ShapeDtypeStruct((B * N, 1), jnp.float32),
        compiler_params=pltpu.CompilerParams(
            dimension_semantics=("parallel",)),
    )(inp, abd, W1, row(b1), row(g1), row(be1), W2, row(b2),
      W3[:M], W3[M:], row(b3), row(g2), row(be2), W4.reshape(1, M))

    out = pl.pallas_call(
        _softmax_kernel,
        out_shape=jax.ShapeDtypeStruct((B, N), jnp.float32),
    )(logits.reshape(B, N))
    return out
